# Initial kernel scaffold; baseline (speedup 1.0000x reference)
import jax
import jax.numpy as jnp
from jax.experimental import pallas as pl


def kernel(x):
    raise NotImplementedError("write your pallas kernel here")


# speedup vs baseline: 1.7260x; 1.7260x over previous
import jax
import jax.numpy as jnp
from jax.experimental import pallas as pl

BATCH = 16384
NUM_CLASSES = 1000
ROW_BLOCK = 512


def _onehot_block(x_ref, o_ref):
    idx = x_ref[...]
    cols = jax.lax.broadcasted_iota(jnp.int32, o_ref.shape, 1)
    o_ref[...] = (cols == idx).astype(jnp.float32)


def kernel(x):
    x = x.astype(jnp.int32)
    return pl.pallas_call(
        _onehot_block,
        grid=(BATCH // ROW_BLOCK,),
        in_specs=[pl.BlockSpec((ROW_BLOCK, 1), lambda i: (i, 0))],
        out_specs=pl.BlockSpec((ROW_BLOCK, NUM_CLASSES), lambda i: (i, 0)),
        out_shape=jax.ShapeDtypeStruct((BATCH, NUM_CLASSES), jnp.float32),
    )(x)


# speedup vs baseline: 1.8608x; 1.0781x over previous
import jax
import jax.numpy as jnp
from jax.experimental import pallas as pl

BATCH = 16384
NUM_CLASSES = 1000
ROW_BLOCK = 1024


def _onehot_block(x_ref, o_ref):
    idx = x_ref[...]
    cols = jax.lax.broadcasted_iota(jnp.int32, o_ref.shape, 1)
    o_ref[...] = (cols == idx).astype(jnp.float32)


def kernel(x):
    x = x.astype(jnp.int32)
    return pl.pallas_call(
        _onehot_block,
        grid=(BATCH // ROW_BLOCK,),
        in_specs=[pl.BlockSpec((ROW_BLOCK, 1), lambda i: (i, 0))],
        out_specs=pl.BlockSpec((ROW_BLOCK, NUM_CLASSES), lambda i: (i, 0)),
        out_shape=jax.ShapeDtypeStruct((BATCH, NUM_CLASSES), jnp.float32),
    )(x)


# speedup vs baseline: 1.9210x; 1.0324x over previous
import jax
import jax.numpy as jnp
from jax.experimental import pallas as pl

BATCH = 16384
NUM_CLASSES = 1000
ROW_BLOCK = 2048


def _onehot_block(x_ref, o_ref):
    idx = x_ref[...]
    cols = jax.lax.broadcasted_iota(jnp.int32, o_ref.shape, 1)
    o_ref[...] = (cols == idx).astype(jnp.float32)


def kernel(x):
    x = x.astype(jnp.int32)
    return pl.pallas_call(
        _onehot_block,
        grid=(BATCH // ROW_BLOCK,),
        in_specs=[pl.BlockSpec((ROW_BLOCK, 1), lambda i: (i, 0))],
        out_specs=pl.BlockSpec((ROW_BLOCK, NUM_CLASSES), lambda i: (i, 0)),
        out_shape=jax.ShapeDtypeStruct((BATCH, NUM_CLASSES), jnp.float32),
    )(x)


# speedup vs baseline: 1.9259x; 1.0025x over previous
import jax
import jax.numpy as jnp
from jax.experimental import pallas as pl

BATCH = 16384
NUM_CLASSES = 1000
ROW_BLOCK = 4096


def _onehot_block(x_ref, o_ref):
    idx = x_ref[...]
    cols = jax.lax.broadcasted_iota(jnp.int32, o_ref.shape, 1)
    o_ref[...] = (cols == idx).astype(jnp.float32)


def kernel(x):
    x = x.astype(jnp.int32)
    return pl.pallas_call(
        _onehot_block,
        grid=(BATCH // ROW_BLOCK,),
        in_specs=[pl.BlockSpec((ROW_BLOCK, 1), lambda i: (i, 0))],
        out_specs=pl.BlockSpec((ROW_BLOCK, NUM_CLASSES), lambda i: (i, 0)),
        out_shape=jax.ShapeDtypeStruct((BATCH, NUM_CLASSES), jnp.float32),
    )(x)
